# baseline (speedup 1.0000x reference)
import functools

import jax
import jax.numpy as jnp
import numpy as np
from jax import lax
from jax.experimental import pallas as pl
from jax.experimental.pallas import tpu as pltpu
from jax.experimental.pallas import tpu_sc as plsc

_BATCH = 4096
_NBINS = 256
_NQ = 8192
_BW = 1.0 / _NBINS
_EPS2 = float(np.spacing(np.finfo(np.float32).eps))

_NC = 2
_NS = 16
_NW = _NC * _NS
_L = 16
_ROWS_PER_W = _BATCH // _NW

_RPC = 2
_CHUNKS = _ROWS_PER_W // _RPC
_IN_ROW = 2 * _NBINS
_TAB_ROW = 3 * _NBINS
_IN_BUF = _RPC * _IN_ROW + _L


def _prep_chunk(in_v, tab_v):
    iota = lax.iota(jnp.int32, _L)
    iota2 = iota * 2

    @plsc.parallel_loop(0, _RPC * _NBINS, step=_L, unroll=4)
    def pass_a(k):
        row = k // _NBINS
        kb = k - row * _NBINS
        idx = iota2 + (row * _IN_ROW + 2 * kb)
        av = plsc.load_gather(in_v, [idx])
        centers = (iota2 + (2 * kb + 1)).astype(jnp.float32) * (0.5 * _BW)
        e = jnp.exp(av * 2.0)
        tnh = 1.0 - 2.0 / (e + 1.0)
        xa = centers + tnh * (0.5 * _BW)
        tab_v[pl.ds(pl.multiple_of(row * _TAB_ROW + kb, _L), _L)] = xa

    @plsc.parallel_loop(0, _RPC * _NBINS, step=_L, unroll=4)
    def pass_b(k):
        row = k // _NBINS
        kb = k - row * _NBINS
        ib = row * _IN_ROW
        tb = row * _TAB_ROW
        off = pl.multiple_of(tb + kb, _L)
        xa_v = tab_v[pl.ds(off, _L)]
        nidx = jnp.minimum(iota + (kb + 1), _NBINS - 1) + tb
        xa_n = plsc.load_gather(tab_v, [nidx])
        tv = plsc.load_gather(in_v, [iota2 + (ib + 2 * kb + 1)])
        tn = plsc.load_gather(in_v, [iota2 + (ib + 2 * kb + 3)])
        dx = xa_n - xa_v
        df = tn - tv
        dx0 = jnp.abs(dx) <= _EPS2
        slope = jnp.where(dx0, 0.0, df / jnp.where(dx0, 1.0, dx))
        inter = jnp.where(dx0, tv, tv - slope * xa_v)
        tab_v[pl.ds(off + _NBINS, _L)] = slope
        tab_v[pl.ds(off + 2 * _NBINS, _L)] = inter

    for row in range(_RPC):
        tb = row * _TAB_ROW
        first = tab_v[pl.ds(tb, _L)]
        tab_v[pl.ds(tb, _L)] = jnp.where(iota == 0, -1.0, first)
        last = tab_v[pl.ds(tb + _NBINS - _L, _L)]
        tab_v[pl.ds(tb + _NBINS - _L, _L)] = jnp.where(iota == _L - 1, 2.0, last)


def _compute_chunk(tab_v, x_v, y_v):
    @plsc.parallel_loop(0, _RPC * _NQ, step=_L, unroll=8)
    def q_body(k):
        base = (k // _NQ) * _TAB_ROW
        off = pl.multiple_of(k, _L)
        xv = x_v[pl.ds(off, _L)]
        j = (xv * float(_NBINS)).astype(jnp.int32) + base
        th = plsc.load_gather(tab_v, [j])
        iL = jnp.where(xv < th, j - 1, j)
        s = plsc.load_gather(tab_v, [iL + _NBINS])
        b = plsc.load_gather(tab_v, [iL + 2 * _NBINS])
        y_v[pl.ds(off, _L)] = b + s * xv


def _interp_body(in_hbm, x_hbm, out_hbm,
                 in0, in1, tab0, tab1, x0, x1, y0, y1,
                 isem0, isem1, osem0, osem1):
    wid = lax.axis_index("s") * _NC + lax.axis_index("c")
    row0 = wid * _ROWS_PER_W
    ins, tabs, xs, ys = (in0, in1), (tab0, tab1), (x0, x1), (y0, y1)
    isems, osems = (isem0, isem1), (osem0, osem1)

    def start_in(c, b):
        r = row0 + c * _RPC
        for i in range(_RPC):
            pltpu.async_copy(
                in_hbm.at[r + i], ins[b].at[pl.ds(i * _IN_ROW, _IN_ROW)], isems[b]
            )
            pltpu.async_copy(
                x_hbm.at[r + i], xs[b].at[pl.ds(i * _NQ, _NQ)], isems[b]
            )

    def wait_in(c, b):
        r = row0 + c * _RPC
        for i in range(_RPC):
            pltpu.make_async_copy(
                in_hbm.at[r + i], ins[b].at[pl.ds(i * _IN_ROW, _IN_ROW)], isems[b]
            ).wait()
            pltpu.make_async_copy(
                x_hbm.at[r + i], xs[b].at[pl.ds(i * _NQ, _NQ)], isems[b]
            ).wait()

    def start_out(c, b):
        r = row0 + c * _RPC
        for i in range(_RPC):
            pltpu.async_copy(
                ys[b].at[pl.ds(i * _NQ, _NQ)], out_hbm.at[r + i], osems[b]
            )

    def wait_out(c, b):
        r = row0 + c * _RPC
        for i in range(_RPC):
            pltpu.make_async_copy(
                ys[b].at[pl.ds(i * _NQ, _NQ)], out_hbm.at[r + i], osems[b]
            ).wait()

    start_in(0, 0)
    start_in(1, 1)

    @pl.loop(0, _CHUNKS, step=2)
    def _chunk_loop(c):
        for b in range(2):
            cb = c + b
            wait_in(cb, b)

            @pl.when(cb >= 2)
            def _wait_out():
                wait_out(cb - 2, b)

            _prep_chunk(ins[b], tabs[b])
            _compute_chunk(tabs[b], xs[b], ys[b])
            start_out(cb, b)

            @pl.when(cb < _CHUNKS - 2)
            def _prefetch():
                start_in(cb + 2, b)

    for b in range(2):
        wait_out(_CHUNKS - 2 + b, b)


@functools.partial(jax.jit, donate_argnums=())
def _interp(inp2, x):
    mesh = plsc.VectorSubcoreMesh(
        core_axis_name="c", subcore_axis_name="s", num_cores=_NC, num_subcores=_NS
    )
    return pl.kernel(
        _interp_body,
        out_type=jax.ShapeDtypeStruct((_BATCH, _NQ), jnp.float32),
        mesh=mesh,
        scratch_types=[
            pltpu.VMEM((_IN_BUF,), jnp.float32),
            pltpu.VMEM((_IN_BUF,), jnp.float32),
            pltpu.VMEM((_RPC * _TAB_ROW,), jnp.float32),
            pltpu.VMEM((_RPC * _TAB_ROW,), jnp.float32),
            pltpu.VMEM((_RPC * _NQ,), jnp.float32),
            pltpu.VMEM((_RPC * _NQ,), jnp.float32),
            pltpu.VMEM((_RPC * _NQ,), jnp.float32),
            pltpu.VMEM((_RPC * _NQ,), jnp.float32),
            pltpu.SemaphoreType.DMA,
            pltpu.SemaphoreType.DMA,
            pltpu.SemaphoreType.DMA,
            pltpu.SemaphoreType.DMA,
        ],
        compiler_params=pltpu.CompilerParams(needs_layout_passes=False),
    )(inp2, x)


def kernel(inputs, x):
    inp2 = inputs.reshape(_BATCH, _IN_ROW)
    return _interp(inp2, x)


# speedup vs baseline: 1.0861x; 1.0861x over previous
import functools

import jax
import jax.numpy as jnp
import numpy as np
from jax import lax
from jax.experimental import pallas as pl
from jax.experimental.pallas import tpu as pltpu
from jax.experimental.pallas import tpu_sc as plsc

_BATCH = 4096
_NBINS = 256
_NQ = 8192
_XMIN = 0.0
_XMAX = 1.0
_BW = (_XMAX - _XMIN) / _NBINS
_EPS2 = float(np.spacing(np.finfo(np.float32).eps))

_NC = 2
_NS = 16
_NW = _NC * _NS
_L = 16
_ROWS_PER_W = _BATCH // _NW


def _prep_body(a_ref, t_ref, out_ref):
    a = a_ref[...]
    t = t_ref[...]
    col = lax.broadcasted_iota(jnp.int32, a.shape, 1).astype(jnp.float32)
    centers = (2.0 * col + 1.0) * (0.5 * _BW)
    xa = jnp.clip(centers + jnp.tanh(a) * (0.5 * _BW), _XMIN, _XMAX)
    xa_r = jnp.concatenate([xa[:, 1:], xa[:, -1:]], axis=1)
    t_r = jnp.concatenate([t[:, 1:], t[:, -1:]], axis=1)
    dx = xa_r - xa
    df = t_r - t
    dx0 = jnp.abs(dx) <= _EPS2
    slope = jnp.where(dx0, 0.0, df / jnp.where(dx0, 1.0, dx))
    inter = jnp.where(dx0, t, t - slope * xa)
    th = jnp.where(col == 0.0, -1.0, jnp.where(col == float(_NBINS - 1), 2.0, xa))
    out_ref[...] = jnp.concatenate([th, slope, inter], axis=1)


_PREP_ROWS = 512


def _prep(a, t):
    return pl.pallas_call(
        _prep_body,
        grid=(_BATCH // _PREP_ROWS,),
        in_specs=[
            pl.BlockSpec((_PREP_ROWS, _NBINS), lambda i: (i, 0)),
            pl.BlockSpec((_PREP_ROWS, _NBINS), lambda i: (i, 0)),
        ],
        out_specs=pl.BlockSpec((_PREP_ROWS, 3 * _NBINS), lambda i: (i, 0)),
        out_shape=jax.ShapeDtypeStruct((_BATCH, 3 * _NBINS), jnp.float32),
    )(a, t)


_RPC = 2
_CHUNKS = _ROWS_PER_W // _RPC


def _compute_chunk(tab_v, x_v, y_v):
    @plsc.parallel_loop(0, _RPC * _NQ, step=_L, unroll=8)
    def q_body(k):
        base = (k // _NQ) * (3 * _NBINS)
        off = pl.multiple_of(k, _L)
        xv = x_v[pl.ds(off, _L)]
        j = (xv * float(_NBINS)).astype(jnp.int32) + base
        th = plsc.load_gather(tab_v, [j])
        iL = jnp.where(xv < th, j - 1, j)
        s = plsc.load_gather(tab_v, [iL + _NBINS])
        b = plsc.load_gather(tab_v, [iL + 2 * _NBINS])
        y_v[pl.ds(off, _L)] = b + s * xv


def _interp_body(tab_hbm, x_hbm, out_hbm,
                 tab0, tab1, x0, x1, y0, y1,
                 isem0, isem1, osem0, osem1):
    wid = lax.axis_index("s") * _NC + lax.axis_index("c")
    row0 = wid * _ROWS_PER_W
    tabs, xs, ys = (tab0, tab1), (x0, x1), (y0, y1)
    isems, osems = (isem0, isem1), (osem0, osem1)

    def start_in(c, b):
        r = row0 + c * _RPC
        for i in range(_RPC):
            pltpu.async_copy(
                tab_hbm.at[r + i],
                tabs[b].at[pl.ds(i * 3 * _NBINS, 3 * _NBINS)],
                isems[b],
            )
            pltpu.async_copy(
                x_hbm.at[r + i], xs[b].at[pl.ds(i * _NQ, _NQ)], isems[b]
            )

    def wait_in(c, b):
        r = row0 + c * _RPC
        for i in range(_RPC):
            pltpu.make_async_copy(
                tab_hbm.at[r + i],
                tabs[b].at[pl.ds(i * 3 * _NBINS, 3 * _NBINS)],
                isems[b],
            ).wait()
            pltpu.make_async_copy(
                x_hbm.at[r + i], xs[b].at[pl.ds(i * _NQ, _NQ)], isems[b]
            ).wait()

    def start_out(c, b):
        r = row0 + c * _RPC
        for i in range(_RPC):
            pltpu.async_copy(
                ys[b].at[pl.ds(i * _NQ, _NQ)], out_hbm.at[r + i], osems[b]
            )

    def wait_out(c, b):
        r = row0 + c * _RPC
        for i in range(_RPC):
            pltpu.make_async_copy(
                ys[b].at[pl.ds(i * _NQ, _NQ)], out_hbm.at[r + i], osems[b]
            ).wait()

    start_in(0, 0)
    start_in(1, 1)

    @pl.loop(0, _CHUNKS, step=2)
    def _chunk_loop(c):
        for b in range(2):
            cb = c + b
            wait_in(cb, b)

            @pl.when(cb >= 2)
            def _wait_out():
                wait_out(cb - 2, b)

            _compute_chunk(tabs[b], xs[b], ys[b])
            start_out(cb, b)

            @pl.when(cb < _CHUNKS - 2)
            def _prefetch():
                start_in(cb + 2, b)

    for b in range(2):
        wait_out(_CHUNKS - 2 + b, b)


@functools.partial(jax.jit, donate_argnums=())
def _interp(tables, x):
    mesh = plsc.VectorSubcoreMesh(
        core_axis_name="c", subcore_axis_name="s", num_cores=_NC, num_subcores=_NS
    )
    return pl.kernel(
        _interp_body,
        out_type=jax.ShapeDtypeStruct((_BATCH, _NQ), jnp.float32),
        mesh=mesh,
        scratch_types=[
            pltpu.VMEM((_RPC * 3 * _NBINS,), jnp.float32),
            pltpu.VMEM((_RPC * 3 * _NBINS,), jnp.float32),
            pltpu.VMEM((_RPC * _NQ,), jnp.float32),
            pltpu.VMEM((_RPC * _NQ,), jnp.float32),
            pltpu.VMEM((_RPC * _NQ,), jnp.float32),
            pltpu.VMEM((_RPC * _NQ,), jnp.float32),
            pltpu.SemaphoreType.DMA,
            pltpu.SemaphoreType.DMA,
            pltpu.SemaphoreType.DMA,
            pltpu.SemaphoreType.DMA,
        ],
        compiler_params=pltpu.CompilerParams(needs_layout_passes=False),
    )(tables, x)


def kernel(inputs, x):
    a = inputs[..., 0]
    t = inputs[..., 1]
    tables = _prep(a, t)
    return _interp(tables, x)
